# Initial kernel scaffold; baseline (speedup 1.0000x reference)
import jax
import jax.numpy as jnp
from jax.experimental import pallas as pl


def kernel(x, edge_index, W, V):
    raise NotImplementedError("write your pallas kernel here")


# speedup vs baseline: 5.3491x; 5.3491x over previous
import functools

import jax
import jax.numpy as jnp
from jax import lax
from jax.experimental import pallas as pl
from jax.experimental.pallas import tpu as pltpu
from jax.experimental.pallas import tpu_sc as plsc

N = 10000
E = 320000
F = 128
R = 64

NC = 2
NS = 16
NW = NC * NS

K = 128
N_PAD = 10240
ROWS_PER_TILE = N_PAD // NS
EPT = 10112
N_CHUNK = EPT // K
E_PAD = EPT * NW

BLK = 1024


def _featurize_kernel(x_ref, w_ref, c_ref):
    feat = jnp.dot(x_ref[...], w_ref[...], preferred_element_type=jnp.float32)
    p = jnp.log(jnp.abs(feat) + 1e-12)
    q = (feat < 0).astype(jnp.float32)
    c_ref[...] = jnp.concatenate([p, q], axis=1)


def _combine_kernel(p_ref, v_ref, o_ref):
    s = p_ref[0] + p_ref[1]
    logabs = s[:, :R]
    negc = s[:, R:]
    sign = 1.0 - 2.0 * jnp.mod(negc, 2.0)
    neigh = sign * jnp.exp(logabs)
    o_ref[...] = lax.dot_general(
        neigh, v_ref[...], (((1,), (1,)), ((), ())),
        preferred_element_type=jnp.float32)


def _make_sc_scatter():
    mesh = plsc.VectorSubcoreMesh(core_axis_name="c", subcore_axis_name="s")

    @functools.partial(
        pl.kernel,
        mesh=mesh,
        out_type=jax.ShapeDtypeStruct((NC * N_PAD, F), jnp.float32),
        scratch_types=[
            pltpu.VMEM((K,), jnp.int32),
            pltpu.VMEM((K,), jnp.int32),
            pltpu.VMEM((K, F), jnp.float32),
            pltpu.VMEM_SHARED((N_PAD, F), jnp.float32),
            pltpu.SemaphoreType.DMA,
        ],
    )
    def sc_scatter(c_hbm, src_hbm, dst_hbm, out_hbm, src_v, dst_v, rows_v,
                   acc, sem):
        cid = lax.axis_index("c")
        sid = lax.axis_index("s")
        wid = sid * NC + cid

        def zero_row(r, carry):
            for j in range(F // 16):
                rows_v[r, pl.ds(j * 16, 16)] = jnp.zeros((16,), jnp.float32)
            return carry

        lax.fori_loop(0, K, zero_row, 0)
        for b in range(ROWS_PER_TILE // K):
            pltpu.sync_copy(rows_v,
                            acc.at[pl.ds(sid * ROWS_PER_TILE + b * K, K)])
        plsc.subcore_barrier()

        ebase = wid * EPT

        def chunk(i, carry):
            off = ebase + i * K
            pltpu.sync_copy(src_hbm.at[pl.ds(off, K)], src_v)
            pltpu.sync_copy(dst_hbm.at[pl.ds(off, K)], dst_v)
            pltpu.async_copy(c_hbm.at[src_v], rows_v, sem).wait()
            pltpu.sync_copy(rows_v, acc.at[dst_v], add=True)
            return carry

        lax.fori_loop(0, N_CHUNK, chunk, 0)
        plsc.subcore_barrier()

        obase = cid * N_PAD + sid * ROWS_PER_TILE
        pltpu.sync_copy(acc.at[pl.ds(sid * ROWS_PER_TILE, ROWS_PER_TILE)],
                        out_hbm.at[pl.ds(obase, ROWS_PER_TILE)])

    return sc_scatter


def kernel(x, edge_index, W, V):
    x_pad = jnp.concatenate(
        [x, jnp.zeros((N_PAD - N, F), jnp.float32)], axis=0)

    c = pl.pallas_call(
        _featurize_kernel,
        grid=(N_PAD // BLK,),
        in_specs=[
            pl.BlockSpec((BLK, F), lambda i: (i, 0)),
            pl.BlockSpec((F, R), lambda i: (0, 0)),
        ],
        out_specs=pl.BlockSpec((BLK, F), lambda i: (i, 0)),
        out_shape=jax.ShapeDtypeStruct((N_PAD, F), jnp.float32),
    )(x_pad, W)

    ei = edge_index.astype(jnp.int32)
    pad = jnp.full((E_PAD - E,), N, jnp.int32)
    src = jnp.concatenate([ei[0], pad])
    dst = jnp.concatenate([ei[1], pad])

    partials = _make_sc_scatter()(c, src, dst)
    partials = partials.reshape(NC, N_PAD, F)

    out_pad = pl.pallas_call(
        _combine_kernel,
        grid=(N_PAD // BLK,),
        in_specs=[
            pl.BlockSpec((NC, BLK, F), lambda i: (0, i, 0)),
            pl.BlockSpec((F, R), lambda i: (0, 0)),
        ],
        out_specs=pl.BlockSpec((BLK, F), lambda i: (i, 0)),
        out_shape=jax.ShapeDtypeStruct((N_PAD, F), jnp.float32),
    )(partials, V)

    return out_pad[:N]
